# baseline (speedup 1.0000x reference)
import jax
import jax.numpy as jnp
from jax import lax
from jax.experimental import pallas as pl
from jax.experimental.pallas import tpu as pltpu
from jax.experimental.pallas import tpu_sc as plsc

L_SIZE = 32
BATCH = 16384
NC, NS, LANES = 2, 16, 16
NW = NC * NS
BPW = BATCH // NW
CHUNK = 128
NCHUNK = BPW // CHUNK


def _mf_body(users_hbm, items_hbm, gb_hbm, ub_hbm, ib_hbm, ue_hbm, ie_hbm,
             out_hbm, idx_u, idx_i, ubv, ibv, uer, ier, outv, gbv, sem):
    wid = lax.axis_index("s") * NC + lax.axis_index("c")
    row0 = wid * NCHUNK

    pltpu.sync_copy(gb_hbm, gbv)
    pltpu.sync_copy(users_hbm.at[pl.ds(row0, NCHUNK)], idx_u)
    pltpu.sync_copy(items_hbm.at[pl.ds(row0, NCHUNK)], idx_i)

    copies = []
    for j in range(NCHUNK):
        sl = pl.ds(j * CHUNK, CHUNK)
        copies.append(pltpu.async_copy(ue_hbm.at[idx_u.at[j]], uer.at[sl], sem))
        copies.append(pltpu.async_copy(ie_hbm.at[idx_i.at[j]], ier.at[sl], sem))
        copies.append(pltpu.async_copy(ub_hbm.at[idx_u.at[j]], ubv.at[sl], sem))
        copies.append(pltpu.async_copy(ib_hbm.at[idx_i.at[j]], ibv.at[sl], sem))
    for c in copies:
        c.wait()

    g = gbv[...]

    def group(gi, carry):
        base = gi * LANES
        bvec = base + lax.iota(jnp.int32, LANES)
        acc = ubv[pl.ds(base, LANES)] + ibv[pl.ds(base, LANES)] + g
        for d in range(L_SIZE):
            dvec = jnp.full((LANES,), d, jnp.int32)
            au = plsc.load_gather(uer, [bvec, dvec])
            ai = plsc.load_gather(ier, [bvec, dvec])
            acc = acc + au * ai
        outv[pl.ds(base, LANES)] = acc
        return carry

    lax.fori_loop(0, BPW // LANES, group, 0)
    pltpu.sync_copy(outv, out_hbm.at[pl.ds(wid * BPW, BPW)])


_mf = pl.kernel(
    _mf_body,
    out_type=jax.ShapeDtypeStruct((BATCH,), jnp.float32),
    mesh=plsc.VectorSubcoreMesh(core_axis_name="c", subcore_axis_name="s"),
    compiler_params=pltpu.CompilerParams(
        needs_layout_passes=False, use_tc_tiling_on_sc=False),
    scratch_types=[
        pltpu.VMEM((NCHUNK, CHUNK), jnp.int32),
        pltpu.VMEM((NCHUNK, CHUNK), jnp.int32),
        pltpu.VMEM((BPW,), jnp.float32),
        pltpu.VMEM((BPW,), jnp.float32),
        pltpu.VMEM((BPW, L_SIZE), jnp.float32),
        pltpu.VMEM((BPW, L_SIZE), jnp.float32),
        pltpu.VMEM((BPW,), jnp.float32),
        pltpu.VMEM((LANES,), jnp.float32),
        pltpu.SemaphoreType.DMA,
    ],
)


def kernel(users, items, globalBias, uBias, itemBias, uEmbed, itemEmbed):
    users2d = users.astype(jnp.int32).reshape(NW * NCHUNK, CHUNK)
    items2d = items.astype(jnp.int32).reshape(NW * NCHUNK, CHUNK)
    gb16 = jnp.broadcast_to(globalBias.reshape(1), (LANES,))
    out = _mf(users2d, items2d, gb16, uBias.reshape(-1),
              itemBias.reshape(-1), uEmbed, itemEmbed)
    return out.reshape(-1, 1)


# speedup vs baseline: 2.4300x; 2.4300x over previous
import jax
import jax.numpy as jnp
from jax import lax
from jax.experimental import pallas as pl
from jax.experimental.pallas import tpu as pltpu
from jax.experimental.pallas import tpu_sc as plsc

L_SIZE = 32
BATCH = 16384
NC, NS, LANES = 2, 16, 16
NW = NC * NS
BPW = BATCH // NW
CHUNK = 128
NCHUNK = BPW // CHUNK
NGRP = BPW // LANES
NOCT = 4


def _fire_tiles(ue_hbm, ie_hbm, ueb, ieb, idx_u, idx_i, sem, o, gi, slot):
    base = gi * LANES
    j = base // CHUNK
    k = base % CHUNK
    uvec = idx_u[j, pl.ds(k, LANES)]
    ivec = idx_i[j, pl.ds(k, LANES)]
    ublk = lax.shift_left(lax.shift_right_logical(uvec, 7), 7)
    iblk = lax.shift_left(lax.shift_right_logical(ivec, 7), 7)
    for p in range(LANES):
        uoff = pl.multiple_of(ublk[p], 128)
        ioff = pl.multiple_of(iblk[p], 128)
        pltpu.async_copy(
            ue_hbm.at[o, :, pl.ds(uoff, 128)], ueb.at[slot, p], sem)
        pltpu.async_copy(
            ie_hbm.at[o, :, pl.ds(ioff, 128)], ieb.at[slot, p], sem)


def _drain_tiles(ue_hbm, ueb, sem):
    for _ in range(2 * LANES):
        pltpu.make_async_copy(
            ue_hbm.at[0, :, pl.ds(0, 128)], ueb.at[0, 0], sem).wait()


def _mf_body(users_hbm, items_hbm, gb_hbm, ub_hbm, ib_hbm, ue_hbm, ie_hbm,
             out_hbm, idx_u, idx_i, ubv, ibv, ueb, ieb, outv, gbv, sem):
    wid = lax.axis_index("s") * NC + lax.axis_index("c")
    row0 = wid * NCHUNK

    pltpu.sync_copy(gb_hbm, gbv)
    pltpu.sync_copy(users_hbm.at[pl.ds(row0, NCHUNK)], idx_u)
    pltpu.sync_copy(items_hbm.at[pl.ds(row0, NCHUNK)], idx_i)

    bias_copies = []
    for j in range(NCHUNK):
        sl = pl.ds(j * CHUNK, CHUNK)
        bias_copies.append(
            pltpu.async_copy(ub_hbm.at[idx_u.at[j]], ubv.at[sl], sem))
        bias_copies.append(
            pltpu.async_copy(ib_hbm.at[idx_i.at[j]], ibv.at[sl], sem))
    for c in bias_copies:
        c.wait()

    g = gbv[...]

    def init(gi, carry):
        sl = pl.ds(gi * LANES, LANES)
        outv[sl] = ubv[sl] + ibv[sl] + g
        return carry

    lax.fori_loop(0, NGRP, init, 0)

    for o in range(NOCT):
        _fire_tiles(ue_hbm, ie_hbm, ueb, ieb, idx_u, idx_i, sem, o, 0, 0)

        def step(gi, carry, o=o):
            slot = lax.rem(gi, 2)
            nslot = 1 - slot
            _drain_tiles(ue_hbm, ueb, sem)

            @pl.when(gi + 1 < NGRP)
            def _():
                _fire_tiles(ue_hbm, ie_hbm, ueb, ieb, idx_u, idx_i, sem,
                            o, gi + 1, nslot)

            base = gi * LANES
            j = base // CHUNK
            k = base % CHUNK
            pvec = lax.iota(jnp.int32, LANES)
            uvec = idx_u[j, pl.ds(k, LANES)]
            ivec = idx_i[j, pl.ds(k, LANES)]
            cu = uvec & 127
            ci = ivec & 127
            svec = jnp.full((LANES,), 0, jnp.int32) + slot
            acc = jnp.zeros((LANES,), jnp.float32)
            for dl in range(8):
                dvec = jnp.full((LANES,), dl, jnp.int32)
                au = plsc.load_gather(ueb, [svec, pvec, dvec, cu])
                ai = plsc.load_gather(ieb, [svec, pvec, dvec, ci])
                acc = acc + au * ai
            sl = pl.ds(base, LANES)
            outv[sl] = outv[sl] + acc
            return carry

        lax.fori_loop(0, NGRP, step, 0)

    pltpu.sync_copy(outv, out_hbm.at[pl.ds(wid * BPW, BPW)])


_mf = pl.kernel(
    _mf_body,
    out_type=jax.ShapeDtypeStruct((BATCH,), jnp.float32),
    mesh=plsc.VectorSubcoreMesh(core_axis_name="c", subcore_axis_name="s"),
    compiler_params=pltpu.CompilerParams(
        needs_layout_passes=False, use_tc_tiling_on_sc=True),
    scratch_types=[
        pltpu.VMEM((NCHUNK, CHUNK), jnp.int32),
        pltpu.VMEM((NCHUNK, CHUNK), jnp.int32),
        pltpu.VMEM((BPW,), jnp.float32),
        pltpu.VMEM((BPW,), jnp.float32),
        pltpu.VMEM((2, LANES, 8, 128), jnp.float32),
        pltpu.VMEM((2, LANES, 8, 128), jnp.float32),
        pltpu.VMEM((BPW,), jnp.float32),
        pltpu.VMEM((LANES,), jnp.float32),
        pltpu.SemaphoreType.DMA,
    ],
)


def kernel(users, items, globalBias, uBias, itemBias, uEmbed, itemEmbed):
    users2d = users.astype(jnp.int32).reshape(NW * NCHUNK, CHUNK)
    items2d = items.astype(jnp.int32).reshape(NW * NCHUNK, CHUNK)
    gb16 = jnp.broadcast_to(globalBias.reshape(1), (LANES,))
    ue3 = uEmbed.T.reshape(NOCT, 8, -1)
    ie3 = itemEmbed.T.reshape(NOCT, 8, -1)
    out = _mf(users2d, items2d, gb16, uBias.reshape(-1),
              itemBias.reshape(-1), ue3, ie3)
    return out.reshape(-1, 1)


# speedup vs baseline: 2.7467x; 1.1303x over previous
import jax
import jax.numpy as jnp
from jax import lax
from jax.experimental import pallas as pl
from jax.experimental.pallas import tpu as pltpu
from jax.experimental.pallas import tpu_sc as plsc

L_SIZE = 32
BATCH = 16384
NC, NS, LANES = 2, 16, 16
NW = NC * NS
BPW = BATCH // NW
CHUNK = 128
NCHUNK = BPW // CHUNK
NGRP = BPW // LANES
NOCT = 4


def _fire_tiles(ue_hbm, ie_hbm, ueb, ieb, idx_u, idx_i, sem, o, gi, slot):
    base = gi * LANES
    j = base // CHUNK
    k = base % CHUNK
    uvec = idx_u[j, pl.ds(k, LANES)]
    ivec = idx_i[j, pl.ds(k, LANES)]
    ublk = lax.shift_left(lax.shift_right_logical(uvec, 7), 7)
    iblk = lax.shift_left(lax.shift_right_logical(ivec, 7), 7)
    for p in range(LANES):
        uoff = pl.multiple_of(ublk[p], 128)
        ioff = pl.multiple_of(iblk[p], 128)
        pltpu.async_copy(
            ue_hbm.at[o, :, pl.ds(uoff, 128)], ueb.at[slot, p], sem)
        pltpu.async_copy(
            ie_hbm.at[o, :, pl.ds(ioff, 128)], ieb.at[slot, p], sem)


def _drain_tiles(ue_hbm, ueb, sem):
    for _ in range(2 * LANES):
        pltpu.make_async_copy(
            ue_hbm.at[0, :, pl.ds(0, 128)], ueb.at[0, 0], sem).wait()


NSTEP = NOCT * (BPW // LANES)


def _mf_body(users_hbm, items_hbm, gb_hbm, ub_hbm, ib_hbm, ue_hbm, ie_hbm,
             out_hbm, idx_u, idx_i, ubv, ibv, ueb, ieb, outv, gbv,
             sem_b, sem0, sem1):
    wid = lax.axis_index("s") * NC + lax.axis_index("c")
    row0 = wid * NCHUNK

    pltpu.sync_copy(gb_hbm, gbv)
    pltpu.sync_copy(users_hbm.at[pl.ds(row0, NCHUNK)], idx_u)
    pltpu.sync_copy(items_hbm.at[pl.ds(row0, NCHUNK)], idx_i)

    bias_copies = []
    for j in range(NCHUNK):
        sl = pl.ds(j * CHUNK, CHUNK)
        bias_copies.append(
            pltpu.async_copy(ub_hbm.at[idx_u.at[j]], ubv.at[sl], sem_b))
        bias_copies.append(
            pltpu.async_copy(ib_hbm.at[idx_i.at[j]], ibv.at[sl], sem_b))
    for c in bias_copies:
        c.wait()

    g = gbv[...]

    def init(gi, carry):
        sl = pl.ds(gi * LANES, LANES)
        outv[sl] = ubv[sl] + ibv[sl] + g
        return carry

    lax.fori_loop(0, NGRP, init, 0)

    def fire_s(s, slot, sem):
        @pl.when(s < NSTEP)
        def _():
            _fire_tiles(ue_hbm, ie_hbm, ueb, ieb, idx_u, idx_i, sem,
                        lax.div(s, NGRP), lax.rem(s, NGRP), slot)

    def compute_s(s, slot):
        gi = lax.rem(s, NGRP)
        base = gi * LANES
        j = lax.div(base, CHUNK)
        k = lax.rem(base, CHUNK)
        pvec = lax.iota(jnp.int32, LANES)
        uvec = idx_u[j, pl.ds(k, LANES)]
        ivec = idx_i[j, pl.ds(k, LANES)]
        cu = uvec & 127
        ci = ivec & 127
        svec = jnp.full((LANES,), slot, jnp.int32)
        acc = jnp.zeros((LANES,), jnp.float32)
        for dl in range(8):
            dvec = jnp.full((LANES,), dl, jnp.int32)
            au = plsc.load_gather(ueb, [svec, pvec, dvec, cu])
            ai = plsc.load_gather(ieb, [svec, pvec, dvec, ci])
            acc = acc + au * ai
        sl = pl.ds(base, LANES)
        outv[sl] = outv[sl] + acc

    _fire_tiles(ue_hbm, ie_hbm, ueb, ieb, idx_u, idx_i, sem0, 0, 0, 0)

    def steppair(h, carry):
        s0 = 2 * h
        fire_s(s0 + 1, 1, sem1)
        _drain_tiles(ue_hbm, ueb, sem0)
        compute_s(s0, 0)
        fire_s(s0 + 2, 0, sem0)
        _drain_tiles(ue_hbm, ueb, sem1)
        compute_s(s0 + 1, 1)
        return carry

    lax.fori_loop(0, NSTEP // 2, steppair, 0)

    pltpu.sync_copy(outv, out_hbm.at[pl.ds(wid * BPW, BPW)])


_mf = pl.kernel(
    _mf_body,
    out_type=jax.ShapeDtypeStruct((BATCH,), jnp.float32),
    mesh=plsc.VectorSubcoreMesh(core_axis_name="c", subcore_axis_name="s"),
    compiler_params=pltpu.CompilerParams(
        needs_layout_passes=False, use_tc_tiling_on_sc=True),
    scratch_types=[
        pltpu.VMEM((NCHUNK, CHUNK), jnp.int32),
        pltpu.VMEM((NCHUNK, CHUNK), jnp.int32),
        pltpu.VMEM((BPW,), jnp.float32),
        pltpu.VMEM((BPW,), jnp.float32),
        pltpu.VMEM((2, LANES, 8, 128), jnp.float32),
        pltpu.VMEM((2, LANES, 8, 128), jnp.float32),
        pltpu.VMEM((BPW,), jnp.float32),
        pltpu.VMEM((LANES,), jnp.float32),
        pltpu.SemaphoreType.DMA,
        pltpu.SemaphoreType.DMA,
        pltpu.SemaphoreType.DMA,
    ],
)


def kernel(users, items, globalBias, uBias, itemBias, uEmbed, itemEmbed):
    users2d = users.astype(jnp.int32).reshape(NW * NCHUNK, CHUNK)
    items2d = items.astype(jnp.int32).reshape(NW * NCHUNK, CHUNK)
    gb16 = jnp.broadcast_to(globalBias.reshape(1), (LANES,))
    ue3 = uEmbed.T.reshape(NOCT, 8, -1)
    ie3 = itemEmbed.T.reshape(NOCT, 8, -1)
    out = _mf(users2d, items2d, gb16, uBias.reshape(-1),
              itemBias.reshape(-1), ue3, ie3)
    return out.reshape(-1, 1)


# speedup vs baseline: 3.7045x; 1.3487x over previous
import jax
import jax.numpy as jnp
from jax import lax
from jax.experimental import pallas as pl
from jax.experimental.pallas import tpu as pltpu
from jax.experimental.pallas import tpu_sc as plsc

L_SIZE = 32
BATCH = 16384
NC, NS, LANES = 2, 16, 16
NW = NC * NS
BPW = BATCH // NW
CHUNK = 128
NCHUNK = BPW // CHUNK
NGRP = BPW // LANES
NOCT = 4


def _fire_tiles(ue_hbm, ie_hbm, ueb, ieb, idx_u, idx_i, sem, o, gi, slot):
    base = gi * LANES
    j = base // CHUNK
    k = base % CHUNK
    uvec = idx_u[j, pl.ds(k, LANES)]
    ivec = idx_i[j, pl.ds(k, LANES)]
    ublk = lax.shift_left(lax.shift_right_logical(uvec, 7), 7)
    iblk = lax.shift_left(lax.shift_right_logical(ivec, 7), 7)
    for p in range(LANES):
        uoff = pl.multiple_of(ublk[p], 128)
        ioff = pl.multiple_of(iblk[p], 128)
        pltpu.async_copy(
            ue_hbm.at[o, :, pl.ds(uoff, 128)], ueb.at[slot, p], sem)
        pltpu.async_copy(
            ie_hbm.at[o, :, pl.ds(ioff, 128)], ieb.at[slot, p], sem)


def _drain_tiles(ue_hbm, ueb, sem):
    for _ in range(2 * LANES):
        pltpu.make_async_copy(
            ue_hbm.at[0, :, pl.ds(0, 128)], ueb.at[0, 0], sem).wait()


NSTEP = NOCT * (BPW // LANES)


def _mf_body(users_hbm, items_hbm, gb_hbm, ub_hbm, ib_hbm, ue_hbm, ie_hbm,
             out_hbm, idx_u, idx_i, ubv, ibv, ueb, ieb, outv, gbv,
             sem_b, sem0, sem1):
    wid = lax.axis_index("s") * NC + lax.axis_index("c")
    row0 = wid * NCHUNK

    pltpu.sync_copy(gb_hbm, gbv)
    pltpu.sync_copy(users_hbm.at[pl.ds(row0, NCHUNK)], idx_u)
    pltpu.sync_copy(items_hbm.at[pl.ds(row0, NCHUNK)], idx_i)

    bias_copies = []
    for j in range(NCHUNK):
        sl = pl.ds(j * CHUNK, CHUNK)
        bias_copies.append(
            pltpu.async_copy(ub_hbm.at[0].at[idx_u.at[j]], ubv.at[sl], sem_b))
        bias_copies.append(
            pltpu.async_copy(ib_hbm.at[0].at[idx_i.at[j]], ibv.at[sl], sem_b))
    for c in bias_copies:
        c.wait()

    g = gbv[...]

    def init(gi, carry):
        sl = pl.ds(gi * LANES, LANES)
        outv[sl] = ubv[sl] + ibv[sl] + g
        return carry

    lax.fori_loop(0, NGRP, init, 0)

    def fire_s(s, slot, sem):
        @pl.when(s < NSTEP)
        def _():
            _fire_tiles(ue_hbm, ie_hbm, ueb, ieb, idx_u, idx_i, sem,
                        lax.div(s, NGRP), lax.rem(s, NGRP), slot)

    def compute_s(s, slot):
        gi = lax.rem(s, NGRP)
        base = gi * LANES
        j = lax.div(base, CHUNK)
        k = lax.rem(base, CHUNK)
        pvec = lax.iota(jnp.int32, LANES)
        uvec = idx_u[j, pl.ds(k, LANES)]
        ivec = idx_i[j, pl.ds(k, LANES)]
        cu = uvec & 127
        ci = ivec & 127
        svec = jnp.full((LANES,), slot, jnp.int32)
        acc = jnp.zeros((LANES,), jnp.float32)
        for dl in range(8):
            dvec = jnp.full((LANES,), dl, jnp.int32)
            au = plsc.load_gather(ueb, [svec, pvec, dvec, cu])
            ai = plsc.load_gather(ieb, [svec, pvec, dvec, ci])
            acc = acc + au * ai
        sl = pl.ds(base, LANES)
        outv[sl] = outv[sl] + acc

    _fire_tiles(ue_hbm, ie_hbm, ueb, ieb, idx_u, idx_i, sem0, 0, 0, 0)

    def steppair(h, carry):
        s0 = 2 * h
        fire_s(s0 + 1, 1, sem1)
        _drain_tiles(ue_hbm, ueb, sem0)
        compute_s(s0, 0)
        fire_s(s0 + 2, 0, sem0)
        _drain_tiles(ue_hbm, ueb, sem1)
        compute_s(s0 + 1, 1)
        return carry

    lax.fori_loop(0, NSTEP // 2, steppair, 0)

    pltpu.sync_copy(outv, out_hbm.at[pl.ds(wid * BPW, BPW)])


_mf = pl.kernel(
    _mf_body,
    out_type=jax.ShapeDtypeStruct((BATCH,), jnp.float32),
    mesh=plsc.VectorSubcoreMesh(core_axis_name="c", subcore_axis_name="s"),
    compiler_params=pltpu.CompilerParams(
        needs_layout_passes=False, use_tc_tiling_on_sc=True),
    scratch_types=[
        pltpu.VMEM((NCHUNK, CHUNK), jnp.int32),
        pltpu.VMEM((NCHUNK, CHUNK), jnp.int32),
        pltpu.VMEM((BPW,), jnp.float32),
        pltpu.VMEM((BPW,), jnp.float32),
        pltpu.VMEM((2, LANES, 8, 128), jnp.float32),
        pltpu.VMEM((2, LANES, 8, 128), jnp.float32),
        pltpu.VMEM((BPW,), jnp.float32),
        pltpu.VMEM((LANES,), jnp.float32),
        pltpu.SemaphoreType.DMA,
        pltpu.SemaphoreType.DMA,
        pltpu.SemaphoreType.DMA,
    ],
)


def kernel(users, items, globalBias, uBias, itemBias, uEmbed, itemEmbed):
    users2d = users.astype(jnp.int32).reshape(NW * NCHUNK, CHUNK)
    items2d = items.astype(jnp.int32).reshape(NW * NCHUNK, CHUNK)
    gb16 = jnp.broadcast_to(globalBias.reshape(1), (LANES,))
    ue3 = uEmbed.T.reshape(NOCT, 8, -1)
    ie3 = itemEmbed.T.reshape(NOCT, 8, -1)
    out = _mf(users2d, items2d, gb16, uBias.T, itemBias.T, ue3, ie3)
    return out.reshape(-1, 1)


# speedup vs baseline: 4.0533x; 1.0942x over previous
import jax
import jax.numpy as jnp
from jax import lax
from jax.experimental import pallas as pl
from jax.experimental.pallas import tpu as pltpu
from jax.experimental.pallas import tpu_sc as plsc

L_SIZE = 32
BATCH = 16384
NC, NS, LANES = 2, 16, 16
NW = NC * NS
BPW = BATCH // NW
CHUNK = 128
NCHUNK = BPW // CHUNK
NGRP = BPW // LANES
NOCT = 4


def _fire_tiles(ue_hbm, ie_hbm, ueb, ieb, idx_u, idx_i, sem, o, gi, slot):
    base = gi * LANES
    j = base // CHUNK
    k = base % CHUNK
    uvec = idx_u[j, pl.ds(k, LANES)]
    ivec = idx_i[j, pl.ds(k, LANES)]
    ublk = lax.shift_left(lax.shift_right_logical(uvec, 7), 7)
    iblk = lax.shift_left(lax.shift_right_logical(ivec, 7), 7)
    for p in range(LANES):
        uoff = pl.multiple_of(ublk[p], 128)
        ioff = pl.multiple_of(iblk[p], 128)
        pltpu.async_copy(
            ue_hbm.at[o, :, pl.ds(uoff, 128)], ueb.at[slot, p], sem)
        pltpu.async_copy(
            ie_hbm.at[o, :, pl.ds(ioff, 128)], ieb.at[slot, p], sem)


def _drain_tiles(ue_hbm, ueb, sem):
    for _ in range(2 * LANES):
        pltpu.make_async_copy(
            ue_hbm.at[0, :, pl.ds(0, 128)], ueb.at[0, 0], sem).wait()


NSTEP = NOCT * (BPW // LANES)


def _mf_body(users_hbm, items_hbm, gb_hbm, ub_hbm, ib_hbm, ue_hbm, ie_hbm,
             out_hbm, idx_u, idx_i, ubv, ibv, ueb, ieb, outv, gbv,
             sem_b, sem0, sem1, sem2):
    wid = lax.axis_index("s") * NC + lax.axis_index("c")
    row0 = wid * NCHUNK

    pltpu.sync_copy(gb_hbm, gbv)
    pltpu.sync_copy(users_hbm.at[pl.ds(row0, NCHUNK)], idx_u)
    pltpu.sync_copy(items_hbm.at[pl.ds(row0, NCHUNK)], idx_i)

    bias_copies = []
    for j in range(NCHUNK):
        sl = pl.ds(j * CHUNK, CHUNK)
        bias_copies.append(
            pltpu.async_copy(ub_hbm.at[0].at[idx_u.at[j]], ubv.at[sl], sem_b))
        bias_copies.append(
            pltpu.async_copy(ib_hbm.at[0].at[idx_i.at[j]], ibv.at[sl], sem_b))
    for c in bias_copies:
        c.wait()

    g = gbv[...]

    def init(gi, carry):
        sl = pl.ds(gi * LANES, LANES)
        outv[sl] = ubv[sl] + ibv[sl] + g
        return carry

    lax.fori_loop(0, NGRP, init, 0)

    def fire_s(s, slot, sem):
        @pl.when(s < NSTEP)
        def _():
            _fire_tiles(ue_hbm, ie_hbm, ueb, ieb, idx_u, idx_i, sem,
                        lax.div(s, NGRP), lax.rem(s, NGRP), slot)

    def compute_s(s, slot):
        gi = lax.rem(s, NGRP)
        base = gi * LANES
        j = lax.div(base, CHUNK)
        k = lax.rem(base, CHUNK)
        pvec = lax.iota(jnp.int32, LANES)
        uvec = idx_u[j, pl.ds(k, LANES)]
        ivec = idx_i[j, pl.ds(k, LANES)]
        cu = uvec & 127
        ci = ivec & 127
        svec = jnp.full((LANES,), slot, jnp.int32)
        acc = jnp.zeros((LANES,), jnp.float32)
        for dl in range(8):
            dvec = jnp.full((LANES,), dl, jnp.int32)
            au = plsc.load_gather(ueb, [svec, pvec, dvec, cu])
            ai = plsc.load_gather(ieb, [svec, pvec, dvec, ci])
            acc = acc + au * ai
        sl = pl.ds(base, LANES)
        outv[sl] = outv[sl] + acc

    _fire_tiles(ue_hbm, ie_hbm, ueb, ieb, idx_u, idx_i, sem0, 0, 0, 0)
    _fire_tiles(ue_hbm, ie_hbm, ueb, ieb, idx_u, idx_i, sem1, 0, 1, 1)
    sems = (sem0, sem1, sem2)

    def steptriple(h, carry):
        s0 = 3 * h
        for q in range(3):
            fire_s(s0 + q + 2, (q + 2) % 3, sems[(q + 2) % 3])
            _drain_tiles(ue_hbm, ueb, sems[q])
            compute_s(s0 + q, q)
        return carry

    lax.fori_loop(0, (NSTEP - 2) // 3, steptriple, 0)
    for s, q in ((NSTEP - 2, 0), (NSTEP - 1, 1)):
        _drain_tiles(ue_hbm, ueb, sems[q])
        compute_s(jnp.int32(s), q)

    pltpu.sync_copy(outv, out_hbm.at[pl.ds(wid * BPW, BPW)])


_mf = pl.kernel(
    _mf_body,
    out_type=jax.ShapeDtypeStruct((BATCH,), jnp.float32),
    mesh=plsc.VectorSubcoreMesh(core_axis_name="c", subcore_axis_name="s"),
    compiler_params=pltpu.CompilerParams(
        needs_layout_passes=False, use_tc_tiling_on_sc=True),
    scratch_types=[
        pltpu.VMEM((NCHUNK, CHUNK), jnp.int32),
        pltpu.VMEM((NCHUNK, CHUNK), jnp.int32),
        pltpu.VMEM((BPW,), jnp.float32),
        pltpu.VMEM((BPW,), jnp.float32),
        pltpu.VMEM((3, LANES, 8, 128), jnp.float32),
        pltpu.VMEM((3, LANES, 8, 128), jnp.float32),
        pltpu.VMEM((BPW,), jnp.float32),
        pltpu.VMEM((LANES,), jnp.float32),
        pltpu.SemaphoreType.DMA,
        pltpu.SemaphoreType.DMA,
        pltpu.SemaphoreType.DMA,
        pltpu.SemaphoreType.DMA,
    ],
)


def kernel(users, items, globalBias, uBias, itemBias, uEmbed, itemEmbed):
    users2d = users.astype(jnp.int32).reshape(NW * NCHUNK, CHUNK)
    items2d = items.astype(jnp.int32).reshape(NW * NCHUNK, CHUNK)
    gb16 = jnp.broadcast_to(globalBias.reshape(1), (LANES,))
    ue3 = uEmbed.T.reshape(NOCT, 8, -1)
    ie3 = itemEmbed.T.reshape(NOCT, 8, -1)
    out = _mf(users2d, items2d, gb16, uBias.T, itemBias.T, ue3, ie3)
    return out.reshape(-1, 1)


# speedup vs baseline: 4.0644x; 1.0027x over previous
import jax
import jax.numpy as jnp
from jax import lax
from jax.experimental import pallas as pl
from jax.experimental.pallas import tpu as pltpu
from jax.experimental.pallas import tpu_sc as plsc

L_SIZE = 32
BATCH = 16384
NC, NS, LANES = 2, 16, 16
NW = NC * NS
BPW = BATCH // NW
CHUNK = 128
NCHUNK = BPW // CHUNK
NGRP = BPW // LANES
NOCT = 4


def _fire_tiles(ue_hbm, ie_hbm, ueb, ieb, idx_u, idx_i, sem, o, gi, slot):
    base = gi * LANES
    j = base // CHUNK
    k = base % CHUNK
    uvec = idx_u[j, pl.ds(k, LANES)]
    ivec = idx_i[j, pl.ds(k, LANES)]
    ublk = lax.shift_left(lax.shift_right_logical(uvec, 7), 7)
    iblk = lax.shift_left(lax.shift_right_logical(ivec, 7), 7)
    for p in range(LANES):
        uoff = pl.multiple_of(ublk[p], 128)
        ioff = pl.multiple_of(iblk[p], 128)
        pltpu.async_copy(
            ue_hbm.at[o, :, pl.ds(uoff, 128)], ueb.at[slot, p], sem)
        pltpu.async_copy(
            ie_hbm.at[o, :, pl.ds(ioff, 128)], ieb.at[slot, p], sem)


def _drain_tiles(dummy_hbm, ueb, ieb, sem, slot):
    pltpu.make_async_copy(dummy_hbm, ueb.at[slot], sem).wait()
    pltpu.make_async_copy(dummy_hbm, ieb.at[slot], sem).wait()


NSTEP = NOCT * (BPW // LANES)


def _mf_body(users_hbm, items_hbm, gb_hbm, ub_hbm, ib_hbm, ue_hbm, ie_hbm,
             dummy_hbm, out_hbm, idx_u, idx_i, ubv, ibv, ueb, ieb, outv, gbv,
             sem_b, sem0, sem1, sem2):
    wid = lax.axis_index("s") * NC + lax.axis_index("c")
    row0 = wid * NCHUNK

    pltpu.sync_copy(gb_hbm, gbv)
    pltpu.sync_copy(users_hbm.at[pl.ds(row0, NCHUNK)], idx_u)
    pltpu.sync_copy(items_hbm.at[pl.ds(row0, NCHUNK)], idx_i)

    bias_copies = []
    for j in range(NCHUNK):
        sl = pl.ds(j * CHUNK, CHUNK)
        bias_copies.append(
            pltpu.async_copy(ub_hbm.at[0].at[idx_u.at[j]], ubv.at[sl], sem_b))
        bias_copies.append(
            pltpu.async_copy(ib_hbm.at[0].at[idx_i.at[j]], ibv.at[sl], sem_b))
    for c in bias_copies:
        c.wait()

    g = gbv[...]

    def init(gi, carry):
        sl = pl.ds(gi * LANES, LANES)
        outv[sl] = ubv[sl] + ibv[sl] + g
        return carry

    lax.fori_loop(0, NGRP, init, 0)

    def fire_s(s, slot, sem):
        @pl.when(s < NSTEP)
        def _():
            _fire_tiles(ue_hbm, ie_hbm, ueb, ieb, idx_u, idx_i, sem,
                        lax.div(s, NGRP), lax.rem(s, NGRP), slot)

    def compute_s(s, slot):
        gi = lax.rem(s, NGRP)
        base = gi * LANES
        j = lax.div(base, CHUNK)
        k = lax.rem(base, CHUNK)
        pvec = lax.iota(jnp.int32, LANES)
        uvec = idx_u[j, pl.ds(k, LANES)]
        ivec = idx_i[j, pl.ds(k, LANES)]
        cu = uvec & 127
        ci = ivec & 127
        svec = jnp.full((LANES,), slot, jnp.int32)
        acc = jnp.zeros((LANES,), jnp.float32)
        for dl in range(8):
            dvec = jnp.full((LANES,), dl, jnp.int32)
            au = plsc.load_gather(ueb, [svec, pvec, dvec, cu])
            ai = plsc.load_gather(ieb, [svec, pvec, dvec, ci])
            acc = acc + au * ai
        sl = pl.ds(base, LANES)
        outv[sl] = outv[sl] + acc

    _fire_tiles(ue_hbm, ie_hbm, ueb, ieb, idx_u, idx_i, sem0, 0, 0, 0)
    _fire_tiles(ue_hbm, ie_hbm, ueb, ieb, idx_u, idx_i, sem1, 0, 1, 1)
    sems = (sem0, sem1, sem2)

    def steptriple(h, carry):
        s0 = 3 * h
        for q in range(3):
            fire_s(s0 + q + 2, (q + 2) % 3, sems[(q + 2) % 3])
            _drain_tiles(dummy_hbm, ueb, ieb, sems[q], q)
            compute_s(s0 + q, q)
        return carry

    lax.fori_loop(0, (NSTEP - 2) // 3, steptriple, 0)
    for s, q in ((NSTEP - 2, 0), (NSTEP - 1, 1)):
        _drain_tiles(dummy_hbm, ueb, ieb, sems[q], q)
        compute_s(jnp.int32(s), q)

    pltpu.sync_copy(outv, out_hbm.at[pl.ds(wid * BPW, BPW)])


_mf = pl.kernel(
    _mf_body,
    out_type=jax.ShapeDtypeStruct((BATCH,), jnp.float32),
    mesh=plsc.VectorSubcoreMesh(core_axis_name="c", subcore_axis_name="s"),
    compiler_params=pltpu.CompilerParams(
        needs_layout_passes=False, use_tc_tiling_on_sc=True),
    scratch_types=[
        pltpu.VMEM((NCHUNK, CHUNK), jnp.int32),
        pltpu.VMEM((NCHUNK, CHUNK), jnp.int32),
        pltpu.VMEM((BPW,), jnp.float32),
        pltpu.VMEM((BPW,), jnp.float32),
        pltpu.VMEM((3, LANES, 8, 128), jnp.float32),
        pltpu.VMEM((3, LANES, 8, 128), jnp.float32),
        pltpu.VMEM((BPW,), jnp.float32),
        pltpu.VMEM((LANES,), jnp.float32),
        pltpu.SemaphoreType.DMA,
        pltpu.SemaphoreType.DMA,
        pltpu.SemaphoreType.DMA,
        pltpu.SemaphoreType.DMA,
    ],
)


def kernel(users, items, globalBias, uBias, itemBias, uEmbed, itemEmbed):
    users2d = users.astype(jnp.int32).reshape(NW * NCHUNK, CHUNK)
    items2d = items.astype(jnp.int32).reshape(NW * NCHUNK, CHUNK)
    gb16 = jnp.broadcast_to(globalBias.reshape(1), (LANES,))
    ue3 = uEmbed.T.reshape(NOCT, 8, -1)
    ie3 = itemEmbed.T.reshape(NOCT, 8, -1)
    dummy = jnp.zeros((LANES, 8, 128), jnp.float32)
    out = _mf(users2d, items2d, gb16, uBias.T, itemBias.T, ue3, ie3, dummy)
    return out.reshape(-1, 1)


# speedup vs baseline: 4.0797x; 1.0038x over previous
import jax
import jax.numpy as jnp
from jax import lax
from jax.experimental import pallas as pl
from jax.experimental.pallas import tpu as pltpu
from jax.experimental.pallas import tpu_sc as plsc

L_SIZE = 32
BATCH = 16384
NC, NS, LANES = 2, 16, 16
NW = NC * NS
BPW = BATCH // NW
CHUNK = 128
NCHUNK = BPW // CHUNK
NGRP = BPW // LANES
NOCT = 4


def _fire_tiles(ue_hbm, ie_hbm, ueb, ieb, idx_u, idx_i, sem, o, gi, slot):
    base = gi * LANES
    j = base // CHUNK
    k = base % CHUNK
    uvec = idx_u[j, pl.ds(k, LANES)]
    ivec = idx_i[j, pl.ds(k, LANES)]
    ublk = lax.shift_left(lax.shift_right_logical(uvec, 7), 7)
    iblk = lax.shift_left(lax.shift_right_logical(ivec, 7), 7)
    for p in range(LANES):
        uoff = pl.multiple_of(ublk[p], 128)
        ioff = pl.multiple_of(iblk[p], 128)
        pltpu.async_copy(
            ue_hbm.at[o, :, pl.ds(uoff, 128)], ueb.at[slot, p], sem)
        pltpu.async_copy(
            ie_hbm.at[o, :, pl.ds(ioff, 128)], ieb.at[slot, p], sem)


def _drain_tiles(dummy_hbm, ueb, ieb, sem, slot):
    pltpu.make_async_copy(dummy_hbm, ueb.at[slot], sem).wait()
    pltpu.make_async_copy(dummy_hbm, ieb.at[slot], sem).wait()


NSTEP = NOCT * (BPW // LANES)


def _mf_body(users_hbm, items_hbm, gb_hbm, ub_hbm, ib_hbm, ue_hbm, ie_hbm,
             dummy_hbm, out_hbm, idx_u, idx_i, ubv, ibv, ueb, ieb, outv, gbv,
             sem_b, sem0, sem1, sem2):
    wid = lax.axis_index("s") * NC + lax.axis_index("c")
    row0 = wid * NCHUNK

    pltpu.sync_copy(gb_hbm, gbv)
    pltpu.sync_copy(users_hbm.at[pl.ds(row0, NCHUNK)], idx_u)
    pltpu.sync_copy(items_hbm.at[pl.ds(row0, NCHUNK)], idx_i)

    _fire_tiles(ue_hbm, ie_hbm, ueb, ieb, idx_u, idx_i, sem0, 0, 0, 0)
    _fire_tiles(ue_hbm, ie_hbm, ueb, ieb, idx_u, idx_i, sem1, 0, 1, 1)

    bias_copies = []
    for j in range(NCHUNK):
        sl = pl.ds(j * CHUNK, CHUNK)
        bias_copies.append(
            pltpu.async_copy(ub_hbm.at[0].at[idx_u.at[j]], ubv.at[sl], sem_b))
        bias_copies.append(
            pltpu.async_copy(ib_hbm.at[0].at[idx_i.at[j]], ibv.at[sl], sem_b))
    for c in bias_copies:
        c.wait()

    g = gbv[...]

    def init(gi, carry):
        sl = pl.ds(gi * LANES, LANES)
        outv[sl] = ubv[sl] + ibv[sl] + g
        return carry

    lax.fori_loop(0, NGRP, init, 0)

    def fire_s(s, slot, sem):
        @pl.when(s < NSTEP)
        def _():
            _fire_tiles(ue_hbm, ie_hbm, ueb, ieb, idx_u, idx_i, sem,
                        lax.div(s, NGRP), lax.rem(s, NGRP), slot)

    def compute_s(s, slot):
        gi = lax.rem(s, NGRP)
        base = gi * LANES
        j = lax.div(base, CHUNK)
        k = lax.rem(base, CHUNK)
        pvec = lax.iota(jnp.int32, LANES)
        uvec = idx_u[j, pl.ds(k, LANES)]
        ivec = idx_i[j, pl.ds(k, LANES)]
        cu = uvec & 127
        ci = ivec & 127
        svec = jnp.full((LANES,), slot, jnp.int32)
        acc = jnp.zeros((LANES,), jnp.float32)
        for dl in range(8):
            dvec = jnp.full((LANES,), dl, jnp.int32)
            au = plsc.load_gather(ueb, [svec, pvec, dvec, cu])
            ai = plsc.load_gather(ieb, [svec, pvec, dvec, ci])
            acc = acc + au * ai
        sl = pl.ds(base, LANES)
        outv[sl] = outv[sl] + acc

    sems = (sem0, sem1, sem2)

    def steptriple(h, carry):
        s0 = 3 * h
        for q in range(3):
            fire_s(s0 + q + 2, (q + 2) % 3, sems[(q + 2) % 3])
            _drain_tiles(dummy_hbm, ueb, ieb, sems[q], q)
            compute_s(s0 + q, q)
        return carry

    lax.fori_loop(0, (NSTEP - 2) // 3, steptriple, 0)
    for s, q in ((NSTEP - 2, 0), (NSTEP - 1, 1)):
        _drain_tiles(dummy_hbm, ueb, ieb, sems[q], q)
        compute_s(jnp.int32(s), q)

    pltpu.sync_copy(outv, out_hbm.at[pl.ds(wid * BPW, BPW)])


_mf = pl.kernel(
    _mf_body,
    out_type=jax.ShapeDtypeStruct((BATCH,), jnp.float32),
    mesh=plsc.VectorSubcoreMesh(core_axis_name="c", subcore_axis_name="s"),
    compiler_params=pltpu.CompilerParams(
        needs_layout_passes=False, use_tc_tiling_on_sc=True),
    scratch_types=[
        pltpu.VMEM((NCHUNK, CHUNK), jnp.int32),
        pltpu.VMEM((NCHUNK, CHUNK), jnp.int32),
        pltpu.VMEM((BPW,), jnp.float32),
        pltpu.VMEM((BPW,), jnp.float32),
        pltpu.VMEM((3, LANES, 8, 128), jnp.float32),
        pltpu.VMEM((3, LANES, 8, 128), jnp.float32),
        pltpu.VMEM((BPW,), jnp.float32),
        pltpu.VMEM((LANES,), jnp.float32),
        pltpu.SemaphoreType.DMA,
        pltpu.SemaphoreType.DMA,
        pltpu.SemaphoreType.DMA,
        pltpu.SemaphoreType.DMA,
    ],
)


def kernel(users, items, globalBias, uBias, itemBias, uEmbed, itemEmbed):
    users2d = users.astype(jnp.int32).reshape(NW * NCHUNK, CHUNK)
    items2d = items.astype(jnp.int32).reshape(NW * NCHUNK, CHUNK)
    gb16 = jnp.broadcast_to(globalBias.reshape(1), (LANES,))
    ue3 = uEmbed.T.reshape(NOCT, 8, -1)
    ie3 = itemEmbed.T.reshape(NOCT, 8, -1)
    dummy = jnp.zeros((LANES, 8, 128), jnp.float32)
    out = _mf(users2d, items2d, gb16, uBias.T, itemBias.T, ue3, ie3, dummy)
    return out.reshape(-1, 1)
